# baseline (speedup 1.0000x reference)
import jax
import jax.numpy as jnp
from jax.experimental import pallas as pl
from jax.experimental.pallas import tpu as pltpu

VOCAB = 100000
EMBED = 10
CTX = 2
HIDDEN = 128
T = 6400
K = 16
KH = K // 2
PAD = K * T
TB = 12800
JH = PAD // (2 * TB)


def _logits_body(embeds_ref, w1_ref, b1_ref, w2_ref, b2_ref,
                 out_ref, mstat_ref, sstat_ref, m_ref, s_ref, h_ref):
    p = pl.program_id(0)
    j = pl.program_id(1)
    i = p * KH + j

    @pl.when(j == 0)
    def _init():
        e = embeds_ref[...]
        h = jnp.dot(e, w1_ref[...], preferred_element_type=jnp.float32)
        h_ref[...] = jnp.maximum(h + b1_ref[...], 0.0)
        m_ref[...] = jnp.full((1, 1), -jnp.inf, jnp.float32)
        s_ref[...] = jnp.zeros((1, 1), jnp.float32)

    h = h_ref[...]
    logits = jnp.dot(h, w2_ref[...], preferred_element_type=jnp.float32)
    logits = logits + b2_ref[...]
    col = i * T + jax.lax.broadcasted_iota(jnp.int32, (1, T), 1)
    masked = jnp.where(col < VOCAB, logits, -jnp.inf)
    out_ref[...] = masked

    m_old = m_ref[...]
    m_new = jnp.maximum(m_old, jnp.max(masked, keepdims=True))
    s_ref[...] = (s_ref[...] * jnp.exp(m_old - m_new)
                  + jnp.sum(jnp.exp(masked - m_new), keepdims=True))
    m_ref[...] = m_new

    @pl.when(j == KH - 1)
    def _publish():
        mstat_ref[...] = jnp.broadcast_to(m_ref[...], (8, 128))
        sstat_ref[...] = jnp.broadcast_to(s_ref[...], (8, 128))


def _norm_body(logits_ref, mstat_ref, sstat_ref, out_ref):
    ms = mstat_ref[...]
    ss = sstat_ref[...]
    m = jnp.max(ms, axis=0, keepdims=True)
    s = jnp.sum(ss * jnp.exp(ms - m), axis=0, keepdims=True) * 0.125
    logz = m + jnp.log(s)
    out_ref[...] = logits_ref[...] - logz[0:1, 0:1]


def _dense(embeds, W1, b1, W2, b2):
    logits, mstat, sstat = pl.pallas_call(
        _logits_body,
        grid=(2, KH),
        in_specs=[
            pl.BlockSpec((1, CTX * EMBED), lambda p, j: (0, 0)),
            pl.BlockSpec((CTX * EMBED, HIDDEN), lambda p, j: (0, 0)),
            pl.BlockSpec((1, HIDDEN), lambda p, j: (0, 0)),
            pl.BlockSpec((HIDDEN, T), lambda p, j: (0, p * KH + j)),
            pl.BlockSpec((1, T), lambda p, j: (0, p * KH + j)),
        ],
        out_specs=[
            pl.BlockSpec((1, T), lambda p, j: (0, p * KH + j)),
            pl.BlockSpec((8, 128), lambda p, j: (p, 0)),
            pl.BlockSpec((8, 128), lambda p, j: (p, 0)),
        ],
        out_shape=[
            jax.ShapeDtypeStruct((1, PAD), jnp.float32),
            jax.ShapeDtypeStruct((16, 128), jnp.float32),
            jax.ShapeDtypeStruct((16, 128), jnp.float32),
        ],
        scratch_shapes=[
            pltpu.VMEM((1, 1), jnp.float32),
            pltpu.VMEM((1, 1), jnp.float32),
            pltpu.VMEM((1, HIDDEN), jnp.float32),
        ],
        compiler_params=pltpu.CompilerParams(
            dimension_semantics=("parallel", "arbitrary")),
    )(embeds, W1, b1.reshape(1, HIDDEN), W2, b2.reshape(1, VOCAB))

    out = pl.pallas_call(
        _norm_body,
        grid=(2, JH),
        in_specs=[
            pl.BlockSpec((1, TB), lambda p, j: (0, p * JH + j)),
            pl.BlockSpec((16, 128), lambda p, j: (0, 0)),
            pl.BlockSpec((16, 128), lambda p, j: (0, 0)),
        ],
        out_specs=pl.BlockSpec((1, TB), lambda p, j: (0, p * JH + j)),
        out_shape=jax.ShapeDtypeStruct((1, PAD), jnp.float32),
        compiler_params=pltpu.CompilerParams(
            dimension_semantics=("parallel", "arbitrary")),
    )(logits, mstat, sstat)
    return out[:, :VOCAB]


def kernel(x, emb, W1, b1, W2, b2):
    embeds = jnp.take(emb, x, axis=0).reshape(1, CTX * EMBED)
    return _dense(embeds, W1, b1, W2, b2)


# speedup vs baseline: 1.0595x; 1.0595x over previous
import jax
import jax.numpy as jnp
from jax.experimental import pallas as pl
from jax.experimental.pallas import tpu as pltpu

VOCAB = 100000
EMBED = 10
CTX = 2
HIDDEN = 128
T = 8192
K = (VOCAB + T - 1) // T
PAD = K * T
RQ = HIDDEN // 4


def _dense_body(embeds_ref, w1_ref, b1_ref, w2a_ref, w2b_ref, w2c_ref,
                w2d_ref, b2_ref, out_ref, m_ref, s_ref, h_ref):
    i = pl.program_id(0)

    @pl.when(i == 0)
    def _init():
        e = embeds_ref[...]
        h = jnp.dot(e, w1_ref[...], preferred_element_type=jnp.float32)
        h_ref[...] = jnp.maximum(h + b1_ref[...], 0.0)
        m_ref[...] = jnp.full((1, 1), -jnp.inf, jnp.float32)
        s_ref[...] = jnp.zeros((1, 1), jnp.float32)

    h = h_ref[...]
    logits = (jnp.dot(h[:, 0:RQ], w2a_ref[...], preferred_element_type=jnp.float32)
              + jnp.dot(h[:, RQ:2 * RQ], w2b_ref[...], preferred_element_type=jnp.float32)
              + jnp.dot(h[:, 2 * RQ:3 * RQ], w2c_ref[...], preferred_element_type=jnp.float32)
              + jnp.dot(h[:, 3 * RQ:4 * RQ], w2d_ref[...], preferred_element_type=jnp.float32))
    logits = logits + b2_ref[...]
    col = i * T + jax.lax.broadcasted_iota(jnp.int32, (1, T), 1)
    masked = jnp.where(col < VOCAB, logits, -jnp.inf)
    out_ref[0:1, pl.ds(i * T, T)] = masked

    m_old = m_ref[...]
    m_new = jnp.maximum(m_old, jnp.max(masked, keepdims=True))
    s_ref[...] = (s_ref[...] * jnp.exp(m_old - m_new)
                  + jnp.sum(jnp.exp(masked - m_new), keepdims=True))
    m_ref[...] = m_new

    @pl.when(i == K - 1)
    def _finish():
        logz = m_ref[...] + jnp.log(s_ref[...])
        out_ref[...] = out_ref[...] - logz


def _dense(embeds, W1, b1, W2, b2):
    def _rq(q):
        return pl.BlockSpec((RQ, T), lambda i, q=q: (q, i))

    out = pl.pallas_call(
        _dense_body,
        grid=(K,),
        in_specs=[
            pl.BlockSpec((1, CTX * EMBED), lambda i: (0, 0)),
            pl.BlockSpec((CTX * EMBED, HIDDEN), lambda i: (0, 0)),
            pl.BlockSpec((1, HIDDEN), lambda i: (0, 0)),
            _rq(0), _rq(1), _rq(2), _rq(3),
            pl.BlockSpec((1, T), lambda i: (0, i)),
        ],
        out_specs=pl.BlockSpec((1, PAD), lambda i: (0, 0)),
        out_shape=jax.ShapeDtypeStruct((1, PAD), jnp.float32),
        scratch_shapes=[
            pltpu.VMEM((1, 1), jnp.float32),
            pltpu.VMEM((1, 1), jnp.float32),
            pltpu.VMEM((1, HIDDEN), jnp.float32),
        ],
    )(embeds, W1, b1.reshape(1, HIDDEN), W2, W2, W2, W2,
      b2.reshape(1, VOCAB))
    return out[:, :VOCAB]


def kernel(x, emb, W1, b1, W2, b2):
    embeds = jnp.take(emb, x, axis=0).reshape(1, CTX * EMBED)
    return _dense(embeds, W1, b1, W2, b2)
